# baseline (speedup 1.0000x reference)
import functools

import jax
import jax.numpy as jnp
from jax import lax
from jax.experimental import pallas as pl
from jax.experimental.pallas import tpu as pltpu
from jax.experimental.pallas import tpu_sc as plsc

F = 26
V = 100000
D = 64
B = 4096
NC = 2
NS = 16
NW = NC * NS
DPW = D // NW
NG = B // 16
HV = V // 2

_mesh = plsc.VectorSubcoreMesh(core_axis_name="c", subcore_axis_name="s")
_params = pltpu.CompilerParams(needs_layout_passes=False)


@functools.partial(
    pl.kernel,
    mesh=_mesh,
    compiler_params=_params,
    out_type=jax.ShapeDtypeStruct((NW, B), jnp.float32),
    scratch_types=[
        pltpu.VMEM((HV,), jnp.float32),
        pltpu.VMEM((HV,), jnp.float32),
        pltpu.VMEM((B,), jnp.int32),
        pltpu.VMEM((B,), jnp.float32),
        pltpu.VMEM((B,), jnp.float32),
        pltpu.VMEM((B,), jnp.float32),
        pltpu.VMEM((B,), jnp.float32),
        pltpu.SemaphoreType.DMA,
        pltpu.SemaphoreType.DMA,
    ],
)
def _fm_part(emb_hbm, xt_hbm, lin_hbm, out_hbm,
             seg0, seg1, xidx, s0, s1, t, part, sem0, sem1):
    w = lax.axis_index("s") * NC + lax.axis_index("c")
    d0 = w * DPW

    segs = (seg0, seg1)
    sems = (sem0, sem1)
    zero = jnp.zeros((16,), jnp.float32)

    def zero_body(g, _):
        sl = pl.ds(g * 16, 16)
        s0[sl] = zero
        s1[sl] = zero
        t[sl] = zero
        return 0

    lax.fori_loop(0, NG, zero_body, 0)

    units = [(f, dj, h) for f in range(F) for dj in range(DPW) for h in (0, 1)]

    def start(i):
        f, dj, h = units[i]
        bi = i % 2
        return pltpu.async_copy(
            emb_hbm.at[(f * D + d0 + dj) * 2 + h], segs[bi], sems[bi])

    def seg_accum(s_ref, seg, h):
        def body(g, _):
            sl = pl.ds(g * 16, 16)
            xv = xidx[sl]
            if h == 0:
                m = xv < HV
                il = jnp.minimum(xv, HV - 1)
            else:
                m = xv >= HV
                il = jnp.maximum(xv - HV, 0)
            e = jnp.where(m, plsc.load_gather(seg, [il]), 0.0)
            s_ref[sl] = s_ref[sl] + e
            t[sl] = t[sl] + e * e
            return 0
        lax.fori_loop(0, NG, body, 0)

    s_refs = (s0, s1)
    pend = start(0)
    for i, (f, dj, h) in enumerate(units):
        nxt = start(i + 1) if i + 1 < len(units) else None
        pend.wait()
        if dj == 0 and h == 0:
            pltpu.sync_copy(xt_hbm.at[f], xidx)
        seg_accum(s_refs[dj], segs[i % 2], h)
        pend = nxt

    def fm_body(g, _):
        sl = pl.ds(g * 16, 16)
        a, b_, c = s0[sl], s1[sl], t[sl]
        part[sl] = 0.5 * (a * a + b_ * b_ - c)
        return 0

    lax.fori_loop(0, NG, fm_body, 0)

    @pl.when(w < F)
    def _():
        pltpu.sync_copy(xt_hbm.at[w], xidx)
        for h in (0, 1):
            pltpu.sync_copy(lin_hbm.at[w * 2 + h], segs[h % 2])

            def lin_body(g, _, h=h):
                sl = pl.ds(g * 16, 16)
                xv = xidx[sl]
                if h == 0:
                    m = xv < HV
                    il = jnp.minimum(xv, HV - 1)
                else:
                    m = xv >= HV
                    il = jnp.maximum(xv - HV, 0)
                lv = jnp.where(m, plsc.load_gather(segs[h % 2], [il]), 0.0)
                part[sl] = part[sl] + lv
                return 0

            lax.fori_loop(0, NG, lin_body, 0)

    pltpu.sync_copy(part, out_hbm.at[w])


@functools.partial(
    pl.kernel,
    mesh=_mesh,
    compiler_params=_params,
    out_type=jax.ShapeDtypeStruct((B,), jnp.float32),
    scratch_types=[
        pltpu.VMEM((NW, B // NW), jnp.float32),
        pltpu.VMEM((16,), jnp.float32),
        pltpu.VMEM((B // NW,), jnp.float32),
    ],
)
def _fm_combine(parts_hbm, bias_hbm, out_hbm, pbuf, bias_v, obuf):
    w = lax.axis_index("s") * NC + lax.axis_index("c")
    bpw = B // NW
    base = w * bpw
    pltpu.sync_copy(bias_hbm, bias_v)
    pltpu.sync_copy(parts_hbm.at[:, pl.ds(base, bpw)], pbuf)
    bias_vec = bias_v[...]

    def body(g, _):
        acc = bias_vec
        for u in range(NW):
            acc = acc + pbuf[u, pl.ds(g * 16, 16)]
        obuf[pl.ds(g * 16, 16)] = 1.0 / (1.0 + jnp.exp(-acc))
        return 0

    lax.fori_loop(0, bpw // 16, body, 0)
    pltpu.sync_copy(obuf, out_hbm.at[pl.ds(base, bpw)])


def kernel(x, emb_tables, lin_tables, bias):
    emb_t = jnp.swapaxes(emb_tables, 1, 2).reshape(F * D * 2, HV)
    xt = x.T.astype(jnp.int32)
    lin2d = lin_tables.reshape(F * 2, HV)
    bias16 = jnp.broadcast_to(bias, (16,))
    parts = _fm_part(emb_t, xt, lin2d)
    out = _fm_combine(parts, bias16)
    return out.reshape(B, 1)


# speedup vs baseline: 1.0048x; 1.0048x over previous
import functools

import jax
import jax.numpy as jnp
from jax import lax
from jax.experimental import pallas as pl
from jax.experimental.pallas import tpu as pltpu
from jax.experimental.pallas import tpu_sc as plsc

F = 26
V = 100000
D = 64
B = 4096
NC = 2
NS = 16
NW = NC * NS
DPW = D // NW
NG = B // 16
HV = V // 2

_mesh = plsc.VectorSubcoreMesh(core_axis_name="c", subcore_axis_name="s")
_params = pltpu.CompilerParams(needs_layout_passes=False)


@functools.partial(
    pl.kernel,
    mesh=_mesh,
    compiler_params=_params,
    out_type=jax.ShapeDtypeStruct((NW, B), jnp.float32),
    scratch_types=[
        pltpu.VMEM((HV,), jnp.float32),
        pltpu.VMEM((HV,), jnp.float32),
        pltpu.VMEM((B,), jnp.int32),
        pltpu.VMEM((B,), jnp.float32),
        pltpu.VMEM((B,), jnp.float32),
        pltpu.VMEM((B,), jnp.float32),
        pltpu.VMEM((B,), jnp.float32),
        pltpu.SemaphoreType.DMA,
        pltpu.SemaphoreType.DMA,
    ],
)
def _fm_part(emb_hbm, xt_hbm, lin_hbm, out_hbm,
             seg0, seg1, xidx, s0, s1, t, part, sem0, sem1):
    w = lax.axis_index("s") * NC + lax.axis_index("c")
    d0 = w * DPW

    segs = (seg0, seg1)
    sems = (sem0, sem1)
    zero = jnp.zeros((16,), jnp.float32)

    @plsc.parallel_loop(0, NG, unroll=8)
    def _(g):
        sl = pl.ds(g * 16, 16)
        s0[sl] = zero
        s1[sl] = zero
        t[sl] = zero

    def row_start(f, dj, h):
        bi = (dj * 2 + h) % 2
        return pltpu.async_copy(
            emb_hbm.at[(f * D + d0 + dj) * 2 + h], segs[bi], sems[bi])

    def seg_accum(s_ref, seg, h):
        @plsc.parallel_loop(0, NG, unroll=8)
        def _(g):
            sl = pl.ds(g * 16, 16)
            xv = xidx[sl]
            if h == 0:
                m = xv < HV
                il = jnp.minimum(xv, HV - 1)
            else:
                m = xv >= HV
                il = jnp.maximum(xv - HV, 0)
            e = jnp.where(m, plsc.load_gather(seg, [il]), 0.0)
            s_ref[sl] = s_ref[sl] + e
            t[sl] = t[sl] + e * e

    s_refs = (s0, s1)
    row_start(0, 0, 0)

    def field_body(f, _):
        pltpu.make_async_copy(emb_hbm.at[0], segs[0], sems[0]).wait()
        a = row_start(f, 0, 1)
        pltpu.sync_copy(xt_hbm.at[f], xidx)
        seg_accum(s0, segs[0], 0)
        a.wait()
        b = row_start(f, 1, 0)
        seg_accum(s0, segs[1], 1)
        b.wait()
        c = row_start(f, 1, 1)
        seg_accum(s1, segs[0], 0)
        c.wait()

        @pl.when(f < F - 1)
        def _():
            pltpu.async_copy(
                emb_hbm.at[((f + 1) * D + d0) * 2], segs[0], sems[0])

        seg_accum(s1, segs[1], 1)
        return 0

    lax.fori_loop(0, F, field_body, 0)

    @plsc.parallel_loop(0, NG, unroll=8)
    def _(g):
        sl = pl.ds(g * 16, 16)
        a, b_, c = s0[sl], s1[sl], t[sl]
        part[sl] = 0.5 * (a * a + b_ * b_ - c)

    @pl.when(w < F)
    def _():
        pltpu.sync_copy(xt_hbm.at[w], xidx)
        for h in (0, 1):
            pltpu.sync_copy(lin_hbm.at[w * 2 + h], segs[h % 2])

            seg = segs[h % 2]

            @plsc.parallel_loop(0, NG, unroll=8)
            def _(g, h=h, seg=seg):
                sl = pl.ds(g * 16, 16)
                xv = xidx[sl]
                if h == 0:
                    m = xv < HV
                    il = jnp.minimum(xv, HV - 1)
                else:
                    m = xv >= HV
                    il = jnp.maximum(xv - HV, 0)
                lv = jnp.where(m, plsc.load_gather(seg, [il]), 0.0)
                part[sl] = part[sl] + lv

    pltpu.sync_copy(part, out_hbm.at[w])


@functools.partial(
    pl.kernel,
    mesh=_mesh,
    compiler_params=_params,
    out_type=jax.ShapeDtypeStruct((B,), jnp.float32),
    scratch_types=[
        pltpu.VMEM((NW, B // NW), jnp.float32),
        pltpu.VMEM((16,), jnp.float32),
        pltpu.VMEM((B // NW,), jnp.float32),
    ],
)
def _fm_combine(parts_hbm, bias_hbm, out_hbm, pbuf, bias_v, obuf):
    w = lax.axis_index("s") * NC + lax.axis_index("c")
    bpw = B // NW
    base = w * bpw
    pltpu.sync_copy(bias_hbm, bias_v)
    pltpu.sync_copy(parts_hbm.at[:, pl.ds(base, bpw)], pbuf)
    bias_vec = bias_v[...]

    @plsc.parallel_loop(0, bpw // 16, unroll=4)
    def _(g):
        acc = bias_vec
        for u in range(NW):
            acc = acc + pbuf[u, pl.ds(g * 16, 16)]
        obuf[pl.ds(g * 16, 16)] = 1.0 / (1.0 + jnp.exp(-acc))
    pltpu.sync_copy(obuf, out_hbm.at[pl.ds(base, bpw)])


def kernel(x, emb_tables, lin_tables, bias):
    emb_t = jnp.swapaxes(emb_tables, 1, 2).reshape(F * D * 2, HV)
    xt = x.T.astype(jnp.int32)
    lin2d = lin_tables.reshape(F * 2, HV)
    bias16 = jnp.broadcast_to(bias, (16,))
    parts = _fm_part(emb_t, xt, lin2d)
    out = _fm_combine(parts, bias16)
    return out.reshape(B, 1)


# speedup vs baseline: 1.0054x; 1.0006x over previous
import functools

import jax
import jax.numpy as jnp
from jax import lax
from jax.experimental import pallas as pl
from jax.experimental.pallas import tpu as pltpu
from jax.experimental.pallas import tpu_sc as plsc

F = 26
V = 100000
D = 64
B = 4096
NC = 2
NS = 16
NW = NC * NS
DPW = D // NW
NG = B // 16
HV = V // 2

_mesh = plsc.VectorSubcoreMesh(core_axis_name="c", subcore_axis_name="s")
_params = pltpu.CompilerParams(needs_layout_passes=False)


@functools.partial(
    pl.kernel,
    mesh=_mesh,
    compiler_params=_params,
    out_type=jax.ShapeDtypeStruct((NW, B), jnp.float32),
    scratch_types=[
        pltpu.VMEM((HV,), jnp.float32),
        pltpu.VMEM((HV,), jnp.float32),
        pltpu.VMEM((B,), jnp.int32),
        pltpu.VMEM((B,), jnp.float32),
        pltpu.VMEM((B,), jnp.float32),
        pltpu.VMEM((B,), jnp.float32),
        pltpu.VMEM((B,), jnp.float32),
        pltpu.SemaphoreType.DMA,
        pltpu.SemaphoreType.DMA,
    ],
)
def _fm_part(emb_hbm, xt_hbm, lin_hbm, out_hbm,
             seg0, seg1, xidx, s0, s1, t, part, sem0, sem1):
    w = lax.axis_index("s") * NC + lax.axis_index("c")
    d0 = w * DPW

    segs = (seg0, seg1)
    sems = (sem0, sem1)
    zero = jnp.zeros((16,), jnp.float32)

    @plsc.parallel_loop(0, NG, unroll=8)
    def _(g):
        sl = pl.ds(g * 16, 16)
        s0[sl] = zero
        s1[sl] = zero
        t[sl] = zero

    def row_start(f, dj, h):
        bi = (dj * 2 + h) % 2
        return pltpu.async_copy(
            emb_hbm.at[(f * D + d0 + dj) * 2 + h], segs[bi], sems[bi])

    def seg_accum(s_ref, seg, h):
        @plsc.parallel_loop(0, NG, unroll=8)
        def _(g):
            sl = pl.ds(g * 16, 16)
            xv = xidx[sl]
            if h == 0:
                m = xv < HV
                il = jnp.minimum(xv, HV - 1)
            else:
                m = xv >= HV
                il = jnp.maximum(xv - HV, 0)
            e = jnp.where(m, plsc.load_gather(seg, [il], mask=m), 0.0)
            s_ref[sl] = s_ref[sl] + e
            t[sl] = t[sl] + e * e

    s_refs = (s0, s1)
    row_start(0, 0, 0)

    def field_body(f, _):
        pltpu.make_async_copy(emb_hbm.at[0], segs[0], sems[0]).wait()
        a = row_start(f, 0, 1)
        pltpu.sync_copy(xt_hbm.at[f], xidx)
        seg_accum(s0, segs[0], 0)
        a.wait()
        b = row_start(f, 1, 0)
        seg_accum(s0, segs[1], 1)
        b.wait()
        c = row_start(f, 1, 1)
        seg_accum(s1, segs[0], 0)
        c.wait()

        @pl.when(f < F - 1)
        def _():
            pltpu.async_copy(
                emb_hbm.at[((f + 1) * D + d0) * 2], segs[0], sems[0])

        seg_accum(s1, segs[1], 1)
        return 0

    lax.fori_loop(0, F, field_body, 0)

    @plsc.parallel_loop(0, NG, unroll=8)
    def _(g):
        sl = pl.ds(g * 16, 16)
        a, b_, c = s0[sl], s1[sl], t[sl]
        part[sl] = 0.5 * (a * a + b_ * b_ - c)

    @pl.when(w < F)
    def _():
        pltpu.sync_copy(xt_hbm.at[w], xidx)
        for h in (0, 1):
            pltpu.sync_copy(lin_hbm.at[w * 2 + h], segs[h % 2])

            seg = segs[h % 2]

            @plsc.parallel_loop(0, NG, unroll=8)
            def _(g, h=h, seg=seg):
                sl = pl.ds(g * 16, 16)
                xv = xidx[sl]
                if h == 0:
                    m = xv < HV
                    il = jnp.minimum(xv, HV - 1)
                else:
                    m = xv >= HV
                    il = jnp.maximum(xv - HV, 0)
                lv = jnp.where(m, plsc.load_gather(seg, [il], mask=m), 0.0)
                part[sl] = part[sl] + lv

    pltpu.sync_copy(part, out_hbm.at[w])


@functools.partial(
    pl.kernel,
    mesh=_mesh,
    compiler_params=_params,
    out_type=jax.ShapeDtypeStruct((B,), jnp.float32),
    scratch_types=[
        pltpu.VMEM((NW, B // NW), jnp.float32),
        pltpu.VMEM((16,), jnp.float32),
        pltpu.VMEM((B // NW,), jnp.float32),
    ],
)
def _fm_combine(parts_hbm, bias_hbm, out_hbm, pbuf, bias_v, obuf):
    w = lax.axis_index("s") * NC + lax.axis_index("c")
    bpw = B // NW
    base = w * bpw
    pltpu.sync_copy(bias_hbm, bias_v)
    pltpu.sync_copy(parts_hbm.at[:, pl.ds(base, bpw)], pbuf)
    bias_vec = bias_v[...]

    @plsc.parallel_loop(0, bpw // 16, unroll=4)
    def _(g):
        acc = bias_vec
        for u in range(NW):
            acc = acc + pbuf[u, pl.ds(g * 16, 16)]
        obuf[pl.ds(g * 16, 16)] = 1.0 / (1.0 + jnp.exp(-acc))
    pltpu.sync_copy(obuf, out_hbm.at[pl.ds(base, bpw)])


def kernel(x, emb_tables, lin_tables, bias):
    emb_t = jnp.swapaxes(emb_tables, 1, 2).reshape(F * D * 2, HV)
    xt = x.T.astype(jnp.int32)
    lin2d = lin_tables.reshape(F * 2, HV)
    bias16 = jnp.broadcast_to(bias, (16,))
    parts = _fm_part(emb_t, xt, lin2d)
    out = _fm_combine(parts, bias16)
    return out.reshape(B, 1)


# speedup vs baseline: 2.6130x; 2.5989x over previous
import functools

import jax
import jax.numpy as jnp
from jax import lax
from jax.experimental import pallas as pl
from jax.experimental.pallas import tpu as pltpu
from jax.experimental.pallas import tpu_sc as plsc

F = 26
V = 100000
D = 64
B = 4096
NC = 2
NS = 16
NW = NC * NS
DPW = D // NW
NG = B // 16
HV = V // 2

_mesh = plsc.VectorSubcoreMesh(core_axis_name="c", subcore_axis_name="s")
_params = pltpu.CompilerParams(needs_layout_passes=False)


@functools.partial(
    pl.kernel,
    mesh=_mesh,
    compiler_params=_params,
    out_type=jax.ShapeDtypeStruct((NW, B), jnp.float32),
    scratch_types=[
        pltpu.VMEM((V,), jnp.float32),
        pltpu.VMEM((B,), jnp.int32),
        pltpu.VMEM((B,), jnp.float32),
        pltpu.VMEM((B,), jnp.float32),
        pltpu.VMEM((B,), jnp.float32),
        pltpu.VMEM((B,), jnp.float32),
        pltpu.SemaphoreType.DMA,
        pltpu.SemaphoreType.DMA,
    ],
)
def _fm_part(emb_hbm, xt_hbm, lin_hbm, out_hbm,
             row_v, xidx, s0, s1, t, part, sem0, sem1):
    w = lax.axis_index("s") * NC + lax.axis_index("c")
    d0 = w * DPW

    zero = jnp.zeros((16,), jnp.float32)

    @plsc.parallel_loop(0, NG, unroll=8)
    def _(g):
        sl = pl.ds(g * 16, 16)
        s0[sl] = zero
        s1[sl] = zero
        t[sl] = zero

    def row_accum(s_ref):
        @plsc.parallel_loop(0, NG, unroll=8)
        def _(g):
            sl = pl.ds(g * 16, 16)
            e = plsc.load_gather(row_v, [xidx[sl]])
            s_ref[sl] = s_ref[sl] + e
            t[sl] = t[sl] + e * e

    def field_body(f, _):
        pltpu.sync_copy(xt_hbm.at[f], xidx)
        pltpu.sync_copy(emb_hbm.at[f * D + d0], row_v)
        row_accum(s0)
        pltpu.sync_copy(emb_hbm.at[f * D + d0 + 1], row_v)
        row_accum(s1)
        return 0

    lax.fori_loop(0, F, field_body, 0)

    @plsc.parallel_loop(0, NG, unroll=8)
    def _(g):
        sl = pl.ds(g * 16, 16)
        a, b_, c = s0[sl], s1[sl], t[sl]
        part[sl] = 0.5 * (a * a + b_ * b_ - c)

    @pl.when(w < F)
    def _():
        pltpu.sync_copy(xt_hbm.at[w], xidx)
        pltpu.sync_copy(lin_hbm.at[w], row_v)

        @plsc.parallel_loop(0, NG, unroll=8)
        def _(g):
            sl = pl.ds(g * 16, 16)
            part[sl] = part[sl] + plsc.load_gather(row_v, [xidx[sl]])

    pltpu.sync_copy(part, out_hbm.at[w])


@functools.partial(
    pl.kernel,
    mesh=_mesh,
    compiler_params=_params,
    out_type=jax.ShapeDtypeStruct((B,), jnp.float32),
    scratch_types=[
        pltpu.VMEM((NW, B // NW), jnp.float32),
        pltpu.VMEM((16,), jnp.float32),
        pltpu.VMEM((B // NW,), jnp.float32),
    ],
)
def _fm_combine(parts_hbm, bias_hbm, out_hbm, pbuf, bias_v, obuf):
    w = lax.axis_index("s") * NC + lax.axis_index("c")
    bpw = B // NW
    base = w * bpw
    pltpu.sync_copy(bias_hbm, bias_v)
    pltpu.sync_copy(parts_hbm.at[:, pl.ds(base, bpw)], pbuf)
    bias_vec = bias_v[...]

    @plsc.parallel_loop(0, bpw // 16, unroll=4)
    def _(g):
        acc = bias_vec
        for u in range(NW):
            acc = acc + pbuf[u, pl.ds(g * 16, 16)]
        obuf[pl.ds(g * 16, 16)] = 1.0 / (1.0 + jnp.exp(-acc))
    pltpu.sync_copy(obuf, out_hbm.at[pl.ds(base, bpw)])


def kernel(x, emb_tables, lin_tables, bias):
    emb_t = jnp.swapaxes(emb_tables, 1, 2).reshape(F * D, V)
    xt = x.T.astype(jnp.int32)
    lin2d = lin_tables.reshape(F, V)
    bias16 = jnp.broadcast_to(bias, (16,))
    parts = _fm_part(emb_t, xt, lin2d)
    out = _fm_combine(parts, bias16)
    return out.reshape(B, 1)


# speedup vs baseline: 2.6495x; 1.0140x over previous
import functools

import jax
import jax.numpy as jnp
from jax import lax
from jax.experimental import pallas as pl
from jax.experimental.pallas import tpu as pltpu
from jax.experimental.pallas import tpu_sc as plsc

F = 26
V = 100000
D = 64
B = 4096
NC = 2
NS = 16
NW = NC * NS
DPW = D // NW
NG = B // 16
HV = V // 2

_mesh = plsc.VectorSubcoreMesh(core_axis_name="c", subcore_axis_name="s")
_params = pltpu.CompilerParams(needs_layout_passes=False)


@functools.partial(
    pl.kernel,
    mesh=_mesh,
    compiler_params=_params,
    out_type=jax.ShapeDtypeStruct((NW, B), jnp.float32),
    scratch_types=[
        pltpu.VMEM((V,), jnp.float32),
        pltpu.VMEM((B,), jnp.int32),
        pltpu.VMEM((B,), jnp.float32),
        pltpu.VMEM((B,), jnp.float32),
        pltpu.VMEM((B,), jnp.float32),
        pltpu.VMEM((B,), jnp.float32),
        pltpu.SemaphoreType.DMA,
        pltpu.SemaphoreType.DMA,
    ],
)
def _fm_part(emb_hbm, xt_hbm, lin_hbm, out_hbm,
             row_v, xidx, s0, s1, t, part, sem0, sem1):
    w = lax.axis_index("s") * NC + lax.axis_index("c")
    d0 = w * DPW

    zero = jnp.zeros((16,), jnp.float32)

    @plsc.parallel_loop(0, NG, unroll=8)
    def _(g):
        sl = pl.ds(g * 16, 16)
        s0[sl] = zero
        s1[sl] = zero
        t[sl] = zero

    def row_accum(s_ref):
        @plsc.parallel_loop(0, NG, unroll=8)
        def _(g):
            sl = pl.ds(g * 16, 16)
            e = plsc.load_gather(row_v, [xidx[sl]])
            s_ref[sl] = s_ref[sl] + e
            t[sl] = t[sl] + e * e

    def field_body(f, _):
        pltpu.sync_copy(xt_hbm.at[f], xidx)
        pltpu.sync_copy(emb_hbm.at[f * D + d0], row_v)
        row_accum(s0)
        pltpu.sync_copy(emb_hbm.at[f * D + d0 + 1], row_v)
        row_accum(s1)
        return 0

    lax.fori_loop(0, F, field_body, 0)

    @plsc.parallel_loop(0, NG, unroll=8)
    def _(g):
        sl = pl.ds(g * 16, 16)
        a, b_, c = s0[sl], s1[sl], t[sl]
        part[sl] = 0.5 * (a * a + b_ * b_ - c)

    @pl.when(w < F)
    def _():
        pltpu.sync_copy(xt_hbm.at[w], xidx)
        pltpu.sync_copy(lin_hbm.at[w], row_v)

        @plsc.parallel_loop(0, NG, unroll=8)
        def _(g):
            sl = pl.ds(g * 16, 16)
            part[sl] = part[sl] + plsc.load_gather(row_v, [xidx[sl]])

    pltpu.sync_copy(part, out_hbm.at[w])


def _combine_body(parts_ref, bias_ref, out_ref):
    acc = jnp.sum(parts_ref[...], axis=0, keepdims=True) + bias_ref[0, 0]
    out_ref[...] = 1.0 / (1.0 + jnp.exp(-acc))


_fm_combine = pl.pallas_call(
    _combine_body,
    out_shape=jax.ShapeDtypeStruct((1, B), jnp.float32),
)


def kernel(x, emb_tables, lin_tables, bias):
    emb_t = jnp.swapaxes(emb_tables, 1, 2).reshape(F * D, V)
    xt = x.T.astype(jnp.int32)
    lin2d = lin_tables.reshape(F, V)
    parts = _fm_part(emb_t, xt, lin2d)
    out = _fm_combine(parts, bias.reshape(1, 1))
    return out.reshape(B, 1)
